# baseline (speedup 1.0000x reference)
import functools

import jax
import jax.numpy as jnp
from jax import lax
from jax.experimental import pallas as pl
from jax.experimental.pallas import tpu as pltpu
from jax.experimental.pallas import tpu_sc as plsc

_ROWS = 1_000_000
_DIM = 64
_LANES = 16
_WORKERS = 32
_CHUNK_ROWS = 400
_NCHUNKS = _ROWS // _CHUNK_ROWS
_GROUPS = _CHUNK_ROWS // _LANES


def _rsqrt16(t):
    i = plsc.bitcast(t, jnp.int32)
    i = jnp.full((_LANES,), 0x5F3759DF, jnp.int32) - (i >> 1)
    y = plsc.bitcast(i, jnp.float32)
    half_t = t * 0.5
    for _ in range(3):
        y = y * (1.5 - half_t * y * y)
    return y


def _normalize_group(xbuf, sbuf, ybuf, g):
    iota = lax.broadcasted_iota(jnp.int32, (_LANES,), 0)
    base = g * _LANES
    for r in range(_LANES):
        row = base + r
        acc = None
        for j in range(_DIM // _LANES):
            v = xbuf[row, pl.ds(j * _LANES, _LANES)]
            sq = v * v
            acc = sq if acc is None else acc + sq
        sbuf[r, :] = acc
    tot = None
    for c in range(_LANES):
        col = plsc.load_gather(sbuf, [iota, jnp.full((_LANES,), c, jnp.int32)])
        tot = col if tot is None else tot + col
    ybuf[...] = _rsqrt16(tot)
    for r in range(_LANES):
        row = base + r
        scale = plsc.load_gather(ybuf, [jnp.full((_LANES,), r, jnp.int32)])
        for j in range(_DIM // _LANES):
            sl = pl.ds(j * _LANES, _LANES)
            xbuf[row, sl] = xbuf[row, sl] * scale


_RING = 4


def _sc_body(w_hbm, o_hbm, xbuf, sbuf, ybuf, *sems):
    isems, osems = sems[:_RING], sems[_RING:]
    wid = lax.axis_index("s") * 2 + lax.axis_index("c")
    nit = (_NCHUNKS - 1 - wid) // _WORKERS + 1

    def in_src(i):
        return w_hbm.at[pl.ds((wid + i * _WORKERS) * _CHUNK_ROWS, _CHUNK_ROWS)]

    def out_dst(i):
        return o_hbm.at[pl.ds((wid + i * _WORKERS) * _CHUNK_ROWS, _CHUNK_ROWS)]

    for b in range(_RING - 1):
        @pl.when(b < nit)
        def _(b=b):
            pltpu.async_copy(in_src(b), xbuf.at[b], isems[b])

    def outer(t, carry):
        for b in range(_RING):
            i = t * _RING + b

            @pl.when(i < nit)
            def _(b=b, i=i):
                pltpu.make_async_copy(in_src(i), xbuf.at[b], isems[b]).wait()

                def group_step(g, c2):
                    _normalize_group(xbuf.at[b], sbuf, ybuf, g)
                    return c2

                lax.fori_loop(0, _GROUPS, group_step, 0, unroll=False)
                pltpu.async_copy(xbuf.at[b], out_dst(i), osems[b])

                nxt = i + _RING - 1
                nb = (b + _RING - 1) % _RING

                @pl.when(nxt < nit)
                def __(b=b, i=i, nxt=nxt, nb=nb):
                    @pl.when(i >= 1)
                    def ___():
                        pltpu.make_async_copy(
                            xbuf.at[nb], out_dst(i - 1), osems[nb]
                        ).wait()

                    pltpu.async_copy(in_src(nxt), xbuf.at[nb], isems[nb])

        return carry

    lax.fori_loop(0, (nit + _RING - 1) // _RING, outer, 0, unroll=False)

    for b in range(_RING):
        jb = nit - 1 - lax.rem(nit - 1 - b + _RING * 4, _RING)

        @pl.when((jb >= 0) & (b < nit))
        def _(b=b, jb=jb):
            pltpu.make_async_copy(xbuf.at[b], out_dst(jb), osems[b]).wait()


def kernel(weight):
    mesh = plsc.VectorSubcoreMesh(core_axis_name="c", subcore_axis_name="s")
    run = functools.partial(
        pl.kernel,
        mesh=mesh,
        out_type=jax.ShapeDtypeStruct((_ROWS, _DIM), jnp.float32),
        scratch_types=[
            pltpu.VMEM((_RING, _CHUNK_ROWS, _DIM), jnp.float32),
            pltpu.VMEM((_LANES, _LANES), jnp.float32),
            pltpu.VMEM((_LANES,), jnp.float32),
        ]
        + [pltpu.SemaphoreType.DMA] * (2 * _RING),
        compiler_params=pltpu.CompilerParams(
            needs_layout_passes=False, use_tc_tiling_on_sc=False
        ),
    )(_sc_body)
    return run(weight)


# speedup vs baseline: 1.8009x; 1.8009x over previous
import jax
import jax.numpy as jnp
from jax import lax
from jax.experimental import pallas as pl
from jax.experimental.pallas import tpu as pltpu

_ROWS = 1_000_000
_DIM = 64
_BLOCK = 4_000
_NCHUNK = _ROWS // _BLOCK
_S = 5
_T = _NCHUNK // _S


def _chunk(ref, c):
    return ref.at[pl.ds(c * _BLOCK, _BLOCK), :]


def _body(x_hbm, o_hbm, xbuf, obuf, isem, osem):
    ones = jnp.ones((_DIM, _DIM), dtype=jnp.float32)

    for s in range(_S):
        pltpu.make_async_copy(_chunk(x_hbm, s), xbuf.at[s], isem.at[s]).start()

    def outer(t, carry):
        for s in range(_S):
            c = t * _S + s
            pltpu.make_async_copy(_chunk(x_hbm, c), xbuf.at[s], isem.at[s]).wait()

            @pl.when(c >= _S)
            def _(s=s, c=c):
                pltpu.make_async_copy(
                    obuf.at[s], _chunk(o_hbm, c - _S), osem.at[s]
                ).wait()

            x = xbuf[s]
            n = jax.lax.dot(x * x, ones, preferred_element_type=jnp.float32)
            obuf[s] = x * jax.lax.rsqrt(n)
            pltpu.make_async_copy(obuf.at[s], _chunk(o_hbm, c), osem.at[s]).start()

            @pl.when(c + _S < _NCHUNK)
            def __(s=s, c=c):
                pltpu.make_async_copy(
                    _chunk(x_hbm, c + _S), xbuf.at[s], isem.at[s]
                ).start()

        return carry

    lax.fori_loop(0, _T, outer, 0, unroll=False)

    for s in range(_S):
        c = _NCHUNK - _S + s
        pltpu.make_async_copy(obuf.at[s], _chunk(o_hbm, c), osem.at[s]).wait()


def kernel(weight):
    return pl.pallas_call(
        _body,
        in_specs=[pl.BlockSpec(memory_space=pltpu.HBM)],
        out_specs=pl.BlockSpec(memory_space=pltpu.HBM),
        out_shape=jax.ShapeDtypeStruct((_ROWS, _DIM), jnp.float32),
        scratch_shapes=[
            pltpu.VMEM((_S, _BLOCK, _DIM), jnp.float32),
            pltpu.VMEM((_S, _BLOCK, _DIM), jnp.float32),
            pltpu.SemaphoreType.DMA((_S,)),
            pltpu.SemaphoreType.DMA((_S,)),
        ],
    )(weight)
